# baseline (speedup 1.0000x reference)
import functools

import jax
import jax.numpy as jnp
from jax import lax
from jax.experimental import pallas as pl
from jax.experimental.pallas import tpu as pltpu
from jax.experimental.pallas import tpu_sc as plsc

BS = 512


def _tc_body(x_ref, emb_ref, out_ref):
    out_ref[...] = x_ref[...] + emb_ref[...][None, :, :]


def _tc_add(x, emb):
    B, T, D = x.shape
    return pl.pallas_call(
        _tc_body,
        grid=(T // BS,),
        in_specs=[
            pl.BlockSpec((B, BS, D), lambda i: (0, i, 0)),
            pl.BlockSpec((BS, D), lambda i: (i, 0)),
        ],
        out_specs=pl.BlockSpec((B, BS, D), lambda i: (0, i, 0)),
        out_shape=jax.ShapeDtypeStruct((B, T, D), x.dtype),
    )(x, emb)


_NC, _NS, _L = 2, 16, 16
_NW = _NC * _NS
_CH = 8


def _sc_add(x, emb, t_lo, t_sc):
    B, T, D = x.shape
    TW = t_sc // _NW
    NCH = TW // _CH
    mesh = plsc.VectorSubcoreMesh(core_axis_name="c", subcore_axis_name="s")

    @functools.partial(
        pl.kernel,
        mesh=mesh,
        out_type=jax.ShapeDtypeStruct((B, t_sc, D), jnp.float32),
        scratch_types=[
            pltpu.VMEM((2, B, _CH, D), jnp.float32),
            pltpu.VMEM((2, _CH, D), jnp.float32),
            pltpu.SemaphoreType.DMA,
            pltpu.SemaphoreType.DMA,
            pltpu.SemaphoreType.DMA,
            pltpu.SemaphoreType.DMA,
            pltpu.SemaphoreType.DMA,
            pltpu.SemaphoreType.DMA,
        ],
    )
    def k(x_hbm, emb_hbm, out_hbm, x_v, emb_v,
          xs0, xs1, es0, es1, os0, os1):
        x_sem = (xs0, xs1)
        e_sem = (es0, es1)
        o_sem = (os0, os1)
        wid = lax.axis_index("s") * _NC + lax.axis_index("c")
        t0 = wid * TW

        def start_in(c, s):
            src_t = t_lo + t0 + c * _CH
            pltpu.make_async_copy(
                emb_hbm.at[pl.ds(src_t, _CH)], emb_v.at[s], e_sem[s]
            ).start()
            pltpu.make_async_copy(
                x_hbm.at[:, pl.ds(src_t, _CH)], x_v.at[s], x_sem[s]
            ).start()

        def wait_in(c, s):
            pltpu.make_async_copy(
                emb_hbm.at[pl.ds(t_lo, _CH)], emb_v.at[s], e_sem[s]
            ).wait()
            pltpu.make_async_copy(
                x_hbm.at[:, pl.ds(t_lo, _CH)], x_v.at[s], x_sem[s]
            ).wait()

        def start_out(c, s):
            dst_t = t0 + c * _CH
            pltpu.make_async_copy(
                x_v.at[s], out_hbm.at[:, pl.ds(dst_t, _CH)], o_sem[s]
            ).start()

        def wait_out(s):
            pltpu.make_async_copy(
                x_v.at[s], out_hbm.at[:, pl.ds(t0, _CH)], o_sem[s]
            ).wait()

        start_in(0, 0)

        def pair(cp, carry):
            for cc in range(2):
                c = 2 * cp + cc
                s = cc

                @pl.when(c >= 1)
                def _():
                    wait_out(1 - s)

                @pl.when(c + 1 < NCH)
                def _():
                    start_in(c + 1, 1 - s)

                wait_in(c, s)

                ng = D // _L

                @plsc.parallel_loop(0, _CH * ng, unroll=8)
                def _(g):
                    r = g // ng
                    j = g % ng
                    sl = pl.ds(j * _L, _L)
                    e = emb_v[s, r, sl]
                    for b in range(B):
                        x_v[s, b, r, sl] = x_v[s, b, r, sl] + e

                start_out(c, s)
            return carry

        lax.fori_loop(0, NCH // 2, pair, 0)
        wait_out(1)

    return k(x, emb)


T_SC = 512


def kernel(x, embed):
    B, T, D = x.shape
    return _sc_add(x, embed[:T], 0, T)


def _hybrid_kernel(x, embed):
    B, T, D = x.shape
    emb = embed[:T]
    t_tc = T - T_SC
    sc_out = _sc_add(x, emb, t_tc, T_SC)
    tc_out = pl.pallas_call(
        _tc_body,
        grid=(t_tc // BS,),
        in_specs=[
            pl.BlockSpec((B, BS, D), lambda i: (0, i, 0)),
            pl.BlockSpec((BS, D), lambda i: (i, 0)),
        ],
        out_specs=pl.BlockSpec((B, BS, D), lambda i: (0, i, 0)),
        out_shape=jax.ShapeDtypeStruct((B, T, D), x.dtype),
    )(x, emb)
    return lax.dynamic_update_slice(tc_out, sc_out, (0, t_tc, 0))


# speedup vs baseline: 1.0209x; 1.0209x over previous
import functools

import jax
import jax.numpy as jnp
from jax import lax
from jax.experimental import pallas as pl
from jax.experimental.pallas import tpu as pltpu
from jax.experimental.pallas import tpu_sc as plsc

_NC, _NS, _L = 2, 16, 16
_NW = _NC * _NS
_CH = 4


def _sc_add(x, emb):
    B, T, D = x.shape
    TW = T // _NW
    NCH = TW // _CH
    mesh = plsc.VectorSubcoreMesh(core_axis_name="c", subcore_axis_name="s")

    @functools.partial(
        pl.kernel,
        mesh=mesh,
        out_type=jax.ShapeDtypeStruct((B, T, D), jnp.float32),
        scratch_types=[
            pltpu.VMEM((2, B, _CH, D), jnp.float32),
            pltpu.VMEM((2, B, _CH, D), jnp.float32),
            pltpu.VMEM((2, _CH, D), jnp.float32),
            pltpu.SemaphoreType.DMA,
            pltpu.SemaphoreType.DMA,
            pltpu.SemaphoreType.DMA,
            pltpu.SemaphoreType.DMA,
            pltpu.SemaphoreType.DMA,
            pltpu.SemaphoreType.DMA,
        ],
    )
    def k(x_hbm, emb_hbm, out_hbm, x_v, o_v, emb_v,
          xs0, xs1, es0, es1, os0, os1):
        x_sem = (xs0, xs1)
        e_sem = (es0, es1)
        o_sem = (os0, os1)
        wid = lax.axis_index("s") * _NC + lax.axis_index("c")
        t0 = wid * TW

        def start_in(c, s):
            src_t = t0 + c * _CH
            pltpu.make_async_copy(
                emb_hbm.at[pl.ds(src_t, _CH)], emb_v.at[s], e_sem[s]
            ).start()
            pltpu.make_async_copy(
                x_hbm.at[:, pl.ds(src_t, _CH)], x_v.at[s], x_sem[s]
            ).start()

        def wait_in(s):
            pltpu.make_async_copy(
                emb_hbm.at[pl.ds(t0, _CH)], emb_v.at[s], e_sem[s]
            ).wait()
            pltpu.make_async_copy(
                x_hbm.at[:, pl.ds(t0, _CH)], x_v.at[s], x_sem[s]
            ).wait()

        def start_out(c, s):
            dst_t = t0 + c * _CH
            pltpu.make_async_copy(
                o_v.at[s], out_hbm.at[:, pl.ds(dst_t, _CH)], o_sem[s]
            ).start()

        def wait_out(s):
            pltpu.make_async_copy(
                o_v.at[s], out_hbm.at[:, pl.ds(t0, _CH)], o_sem[s]
            ).wait()

        start_in(0, 0)

        def pair(cp, carry):
            for cc in range(2):
                c = 2 * cp + cc
                s = cc

                @pl.when(c + 1 < NCH)
                def _():
                    start_in(c + 1, 1 - s)

                @pl.when(c >= 2)
                def _():
                    wait_out(s)

                wait_in(s)

                ng = D // _L

                @plsc.parallel_loop(0, _CH * ng, unroll=8)
                def _(g):
                    r = g // ng
                    j = g % ng
                    sl = pl.ds(j * _L, _L)
                    e = emb_v[s, r, sl]
                    for b in range(B):
                        o_v[s, b, r, sl] = x_v[s, b, r, sl] + e

                start_out(c, s)
            return carry

        lax.fori_loop(0, NCH // 2, pair, 0)
        wait_out(0)
        wait_out(1)

    return k(x, emb)


def kernel(x, embed):
    T = x.shape[1]
    return _sc_add(x, embed[:T])
